# baseline (speedup 1.0000x reference)
import functools

import jax
import jax.numpy as jnp
from jax import lax
from jax.experimental import pallas as pl
from jax.experimental.pallas import tpu as pltpu
from jax.experimental.pallas import tpu_sc as plsc

NROWS, NCOLS, TOPK = 128, 32768, 2048
L = 16
NB = 256
HIST_U = 4
REF_U = 2
NV_ROW = NCOLS // L
NV_K = TOPK // L
ROWS_PER_W = NROWS // 32
SAMP = 4
SAMP_TARGET = TOPK // SAMP + 64


def kernel(x):
    mesh = plsc.VectorSubcoreMesh(core_axis_name="c", subcore_axis_name="s")

    @functools.partial(
        pl.kernel,
        out_type=jax.ShapeDtypeStruct((NROWS, TOPK), jnp.float32),
        mesh=mesh,
        scratch_types=[
            pltpu.VMEM((NCOLS,), jnp.float32),
            pltpu.VMEM((NCOLS,), jnp.float32),
            pltpu.VMEM((HIST_U * L * NB,), jnp.int32),
            pltpu.VMEM((NB,), jnp.int32),
            pltpu.VMEM((TOPK,), jnp.float32),
        ],
        compiler_params=pltpu.CompilerParams(needs_layout_passes=False),
    )
    def body(x_hbm, out_hbm, row_v, cand_v, hist_v, tot_v, sort_v):
        cid = lax.axis_index("c")
        sid = lax.axis_index("s")
        wid = sid * 2 + cid

        lane = lax.iota(jnp.int32, L)
        lane_base = lane * NB
        ones = jnp.ones((L,), jnp.int32)
        zeros = jnp.zeros((L,), jnp.int32)

        def to_key(v):
            b = lax.bitcast_convert_type(v, jnp.int32)
            sgn = b >> 31
            return lax.bitcast_convert_type(b ^ (sgn | jnp.int32(-(2 ** 31))), jnp.uint32)

        @plsc.parallel_loop(0, HIST_U * NB, unroll=8)
        def _zh(i):
            hist_v[pl.ds(i * L, L)] = zeros

        def digit_select(target_v, banks):
            @plsc.parallel_loop(0, NB // L, unroll=2)
            def _lr(blk):
                acc = zeros
                for l in range(L * banks):
                    seg = pl.ds(l * NB + blk * L, L)
                    acc = acc + hist_v[seg]
                    hist_v[seg] = zeros
                tot_v[pl.ds(blk * L, L)] = acc

            s_blk = zeros
            for kk in range(L):
                s_blk = s_blk + plsc.load_gather(tot_v, [lane * L + kk])
            rs = lax.rev(s_blk, (0,))
            rc = plsc.cumsum(rs)
            fb = plsc.all_reduce_ffs(rc >= target_v)
            above_b = rc[fb] - rs[fb]
            blk_sel = (L - 1) - fb
            t2 = plsc.load_gather(tot_v, [blk_sel * L + lane])
            t2r = lax.rev(t2, (0,))
            c2 = plsc.cumsum(t2r) + above_b
            f2 = plsc.all_reduce_ffs(c2 >= target_v)
            above = c2[f2] - t2r[f2]
            bsel = blk_sel * L + (L - 1) - f2
            return bsel, above

        @pl.loop(0, ROWS_PER_W)
        def _row(rl):
            r = wid * ROWS_PER_W + rl
            pltpu.sync_copy(x_hbm.at[r], row_v)

            @plsc.parallel_loop(0, NV_ROW // SAMP, unroll=8)
            def _h1(i):
                b = lax.bitcast_convert_type(row_v[pl.ds(i * (L * SAMP), L)], jnp.int32)
                dig = (b >> 24) ^ ((b >> 31) | jnp.int32(0x80))
                bank = (i & (HIST_U - 1)) * (L * NB)
                plsc.addupdate_scatter(
                    hist_v, [jnp.broadcast_to(bank, (L,)) + lane_base + dig], ones)

            topk_v = jnp.broadcast_to(jnp.int32(TOPK), (L,))
            samp_v = jnp.broadcast_to(jnp.int32(SAMP_TARGET), (L,))
            bhat, _unused = digit_select(samp_v, HIST_U)

            thr_u = bhat.astype(jnp.uint32) << jnp.uint32(24)
            thr_bits = jnp.where(thr_u >= jnp.uint32(0x80000000),
                                 thr_u ^ jnp.uint32(0x80000000), ~thr_u)
            thr_f = lax.bitcast_convert_type(thr_bits, jnp.float32)
            thr_f = jnp.where(bhat == 0, jnp.float32(-jnp.inf), thr_f)

            @plsc.parallel_loop(0, NV_ROW, unroll=8, carry=zeros)
            def compact(i, cnt):
                v = row_v[pl.ds(i * L, L)]
                mk = v >= thr_f
                plsc.store_scatter(cand_v, [(cnt << 4) + lane], v, mask=mk)
                return cnt + jnp.where(mk, 1, 0)

            cnt_v = compact

            def _redo():
                @plsc.parallel_loop(0, NV_ROW, unroll=8, carry=zeros)
                def _call(i, cnt):
                    v = row_v[pl.ds(i * L, L)]
                    plsc.store_scatter(cand_v, [(cnt << 4) + lane], v)
                    return cnt + 1

                return _call

            cnt_v = lax.cond(jnp.sum(cnt_v) < TOPK, _redo, lambda: cnt_v)
            mx = jnp.max(cnt_v)

            @plsc.parallel_loop(0, mx, unroll=REF_U)
            def _hc(i):
                b = lax.bitcast_convert_type(cand_v[pl.ds(i * L, L)], jnp.int32)
                dig = (b >> 24) ^ ((b >> 31) | jnp.int32(0x80))
                valid = jnp.broadcast_to(i, (L,)) < cnt_v
                bank = (i & (REF_U - 1)) * (L * NB)
                plsc.addupdate_scatter(
                    hist_v, [jnp.broadcast_to(bank, (L,)) + lane_base + dig],
                    ones, mask=valid)

            b1, n_hi = digit_select(topk_v, REF_U)

            kth = topk_v - n_hi
            tkey = b1.astype(jnp.uint32) << jnp.uint32(24)
            for s in (16, 8, 0):
                tpref_v = tkey >> jnp.uint32(s + 8)

                @plsc.parallel_loop(0, mx, unroll=REF_U)
                def _hr(i):
                    u = to_key(cand_v[pl.ds(i * L, L)])
                    valid = jnp.broadcast_to(i, (L,)) < cnt_v
                    mk = jnp.logical_and(valid, (u >> jnp.uint32(s + 8)) == tpref_v)
                    dig = ((u >> jnp.uint32(s)) & jnp.uint32(0xFF)).astype(jnp.int32)
                    bank = (i & (REF_U - 1)) * (L * NB)
                    plsc.addupdate_scatter(
                        hist_v, [jnp.broadcast_to(bank, (L,)) + lane_base + dig],
                        ones, mask=mk)

                bl, above = digit_select(kth, REF_U)
                tkey = tkey | (bl.astype(jnp.uint32) << jnp.uint32(s))
                kth = kth - above

            tbits = jnp.where(tkey >= jnp.uint32(0x80000000),
                              tkey ^ jnp.uint32(0x80000000), ~tkey)
            t_f = lax.bitcast_convert_type(tbits, jnp.float32)

            @plsc.parallel_loop(0, NV_K, unroll=8)
            def _fill(i):
                sort_v[pl.ds(i * L, L)] = t_f

            @plsc.parallel_loop(0, mx, unroll=2, carry=zeros)
            def _fc(i, offv):
                v = cand_v[pl.ds(i * L, L)]
                valid = jnp.broadcast_to(i, (L,)) < cnt_v
                mk = jnp.logical_and(v > t_f, valid)
                pos = offv + plsc.cumsum(jnp.where(mk, 1, 0)) - 1
                plsc.store_scatter(sort_v, [pos], v, mask=mk)
                return offv + plsc.all_reduce_population_count(mk)

            def vsort_sweep():
                @plsc.parallel_loop(0, NV_K, unroll=4)
                def _vs(i):
                    v = sort_v[pl.ds(i * L, L)]
                    sk, _ = plsc.sort_key_val(v, v, descending=True)
                    sort_v[pl.ds(i * L, L)] = sk

            vsort_sweep()
            run = 1
            while run < NV_K:
                lg = run.bit_length() - 1

                @plsc.parallel_loop(0, NV_K // 2, unroll=4)
                def _ma(t, run=run, lg=lg):
                    base = (t >> lg) * (2 * run)
                    i = t & (run - 1)
                    p = (base + i) * L
                    q = (base + 2 * run - 1 - i) * L
                    a = sort_v[pl.ds(p, L)]
                    b = lax.rev(sort_v[pl.ds(q, L)], (0,))
                    sort_v[pl.ds(p, L)] = jnp.maximum(a, b)
                    sort_v[pl.ds(q, L)] = lax.rev(jnp.minimum(a, b), (0,))

                d = run // 2
                while d >= 1:
                    lgd = d.bit_length() - 1

                    @plsc.parallel_loop(0, NV_K // 2, unroll=4)
                    def _mb(t, d=d, lgd=lgd):
                        p = ((t >> lgd) * (2 * d) + (t & (d - 1))) * L
                        q = p + d * L
                        a = sort_v[pl.ds(p, L)]
                        b = sort_v[pl.ds(q, L)]
                        sort_v[pl.ds(p, L)] = jnp.maximum(a, b)
                        sort_v[pl.ds(q, L)] = jnp.minimum(a, b)

                    d //= 2
                vsort_sweep()
                run *= 2

            pltpu.sync_copy(sort_v, out_hbm.at[r])

    return body(x)


# speedup vs baseline: 1.0241x; 1.0241x over previous
import functools

import jax
import jax.numpy as jnp
from jax import lax
from jax.experimental import pallas as pl
from jax.experimental.pallas import tpu as pltpu
from jax.experimental.pallas import tpu_sc as plsc

NROWS, NCOLS, TOPK = 128, 32768, 2048
L = 16
NB = 256
HIST_U = 4
REF_U = 2
NV_ROW = NCOLS // L
NV_K = TOPK // L
ROWS_PER_W = NROWS // 32


def kernel(x):
    mesh = plsc.VectorSubcoreMesh(core_axis_name="c", subcore_axis_name="s")

    @functools.partial(
        pl.kernel,
        out_type=jax.ShapeDtypeStruct((NROWS, TOPK), jnp.float32),
        mesh=mesh,
        scratch_types=[
            pltpu.VMEM((NCOLS,), jnp.float32),
            pltpu.VMEM((NCOLS,), jnp.float32),
            pltpu.VMEM((HIST_U * L * NB,), jnp.int32),
            pltpu.VMEM((NB,), jnp.int32),
            pltpu.VMEM((TOPK,), jnp.float32),
        ],
        compiler_params=pltpu.CompilerParams(needs_layout_passes=False),
    )
    def body(x_hbm, out_hbm, row_v, cand_v, hist_v, tot_v, sort_v):
        cid = lax.axis_index("c")
        sid = lax.axis_index("s")
        wid = sid * 2 + cid

        lane = lax.iota(jnp.int32, L)
        lane_base = lane * NB
        ones = jnp.ones((L,), jnp.int32)
        zeros = jnp.zeros((L,), jnp.int32)

        def to_key(v):
            b = lax.bitcast_convert_type(v, jnp.int32)
            sgn = b >> 31
            return lax.bitcast_convert_type(b ^ (sgn | jnp.int32(-(2 ** 31))), jnp.uint32)

        @plsc.parallel_loop(0, HIST_U * NB, unroll=8)
        def _zh(i):
            hist_v[pl.ds(i * L, L)] = zeros

        def digit_select(target_v, banks):
            @plsc.parallel_loop(0, NB // L, unroll=2)
            def _lr(blk):
                acc = zeros
                for l in range(L * banks):
                    seg = pl.ds(l * NB + blk * L, L)
                    acc = acc + hist_v[seg]
                    hist_v[seg] = zeros
                tot_v[pl.ds(blk * L, L)] = acc

            s_blk = zeros
            for kk in range(L):
                s_blk = s_blk + plsc.load_gather(tot_v, [lane * L + kk])
            rs = lax.rev(s_blk, (0,))
            rc = plsc.cumsum(rs)
            fb = plsc.all_reduce_ffs(rc >= target_v)
            above_b = rc[fb] - rs[fb]
            blk_sel = (L - 1) - fb
            t2 = plsc.load_gather(tot_v, [blk_sel * L + lane])
            t2r = lax.rev(t2, (0,))
            c2 = plsc.cumsum(t2r) + above_b
            f2 = plsc.all_reduce_ffs(c2 >= target_v)
            above = c2[f2] - t2r[f2]
            bsel = blk_sel * L + (L - 1) - f2
            return bsel, above

        @pl.loop(0, ROWS_PER_W)
        def _row(rl):
            r = wid * ROWS_PER_W + rl
            pltpu.sync_copy(x_hbm.at[r], row_v)

            @plsc.parallel_loop(0, NV_ROW, unroll=8)
            def _h1(i):
                b = lax.bitcast_convert_type(row_v[pl.ds(i * L, L)], jnp.int32)
                dig = (b >> 24) ^ ((b >> 31) | jnp.int32(0x80))
                bank = (i & (HIST_U - 1)) * (L * NB)
                plsc.addupdate_scatter(
                    hist_v, [jnp.broadcast_to(bank, (L,)) + lane_base + dig], ones)

            topk_v = jnp.broadcast_to(jnp.int32(TOPK), (L,))
            b1, n_hi = digit_select(topk_v, HIST_U)

            thr_u = b1.astype(jnp.uint32) << jnp.uint32(24)
            thr_bits = jnp.where(thr_u >= jnp.uint32(0x80000000),
                                 thr_u ^ jnp.uint32(0x80000000), ~thr_u)
            thr_f = lax.bitcast_convert_type(thr_bits, jnp.float32)
            thr_f = jnp.where(b1 == 0, jnp.float32(-jnp.inf), thr_f)

            @plsc.parallel_loop(0, NV_ROW, unroll=8, carry=zeros)
            def compact(i, cnt):
                v = row_v[pl.ds(i * L, L)]
                mk = v >= thr_f
                plsc.store_scatter(cand_v, [(cnt << 4) + lane], v, mask=mk)
                return cnt + jnp.where(mk, 1, 0)

            cnt_v = compact
            mx = jnp.max(cnt_v)

            kth = topk_v - n_hi
            tkey = b1.astype(jnp.uint32) << jnp.uint32(24)
            for s in (16, 8, 0):
                tpref_v = tkey >> jnp.uint32(s + 8)

                @plsc.parallel_loop(0, mx, unroll=REF_U)
                def _hr(i):
                    u = to_key(cand_v[pl.ds(i * L, L)])
                    valid = jnp.broadcast_to(i, (L,)) < cnt_v
                    mk = jnp.logical_and(valid, (u >> jnp.uint32(s + 8)) == tpref_v)
                    dig = ((u >> jnp.uint32(s)) & jnp.uint32(0xFF)).astype(jnp.int32)
                    bank = (i & (REF_U - 1)) * (L * NB)
                    plsc.addupdate_scatter(
                        hist_v, [jnp.broadcast_to(bank, (L,)) + lane_base + dig],
                        ones, mask=mk)

                bl, above = digit_select(kth, REF_U)
                tkey = tkey | (bl.astype(jnp.uint32) << jnp.uint32(s))
                kth = kth - above

            tbits = jnp.where(tkey >= jnp.uint32(0x80000000),
                              tkey ^ jnp.uint32(0x80000000), ~tkey)
            t_f = lax.bitcast_convert_type(tbits, jnp.float32)

            @plsc.parallel_loop(0, NV_K, unroll=8)
            def _fill(i):
                sort_v[pl.ds(i * L, L)] = t_f

            @plsc.parallel_loop(0, mx, unroll=2, carry=zeros)
            def _fc(i, offv):
                v = cand_v[pl.ds(i * L, L)]
                valid = jnp.broadcast_to(i, (L,)) < cnt_v
                mk = jnp.logical_and(v > t_f, valid)
                pos = offv + plsc.cumsum(jnp.where(mk, 1, 0)) - 1
                plsc.store_scatter(sort_v, [pos], v, mask=mk)
                return offv + plsc.all_reduce_population_count(mk)

            def vsort_sweep():
                @plsc.parallel_loop(0, NV_K, unroll=4)
                def _vs(i):
                    v = sort_v[pl.ds(i * L, L)]
                    sk, _ = plsc.sort_key_val(v, v, descending=True)
                    sort_v[pl.ds(i * L, L)] = sk

            vsort_sweep()
            run = 1
            while run < NV_K:
                lg = run.bit_length() - 1

                @plsc.parallel_loop(0, NV_K // 2, unroll=4)
                def _ma(t, run=run, lg=lg):
                    base = (t >> lg) * (2 * run)
                    i = t & (run - 1)
                    p = (base + i) * L
                    q = (base + 2 * run - 1 - i) * L
                    a = sort_v[pl.ds(p, L)]
                    b = lax.rev(sort_v[pl.ds(q, L)], (0,))
                    sort_v[pl.ds(p, L)] = jnp.maximum(a, b)
                    sort_v[pl.ds(q, L)] = lax.rev(jnp.minimum(a, b), (0,))

                d = run // 2
                while d >= 1:
                    lgd = d.bit_length() - 1

                    @plsc.parallel_loop(0, NV_K // 2, unroll=4)
                    def _mb(t, d=d, lgd=lgd):
                        p = ((t >> lgd) * (2 * d) + (t & (d - 1))) * L
                        q = p + d * L
                        a = sort_v[pl.ds(p, L)]
                        b = sort_v[pl.ds(q, L)]
                        sort_v[pl.ds(p, L)] = jnp.maximum(a, b)
                        sort_v[pl.ds(q, L)] = jnp.minimum(a, b)

                    d //= 2
                vsort_sweep()
                run *= 2

            pltpu.sync_copy(sort_v, out_hbm.at[r])

    return body(x)


# speedup vs baseline: 1.0283x; 1.0042x over previous
import functools

import jax
import jax.numpy as jnp
from jax import lax
from jax.experimental import pallas as pl
from jax.experimental.pallas import tpu as pltpu
from jax.experimental.pallas import tpu_sc as plsc

NROWS, NCOLS, TOPK = 128, 32768, 2048
L = 16
NB = 256
HIST_U = 4
REF_U = 2
NV_ROW = NCOLS // L
NV_K = TOPK // L
ROWS_PER_W = NROWS // 32


def kernel(x):
    mesh = plsc.VectorSubcoreMesh(core_axis_name="c", subcore_axis_name="s")

    @functools.partial(
        pl.kernel,
        out_type=jax.ShapeDtypeStruct((NROWS, TOPK), jnp.float32),
        mesh=mesh,
        scratch_types=[
            pltpu.VMEM((NCOLS,), jnp.float32),
            pltpu.VMEM((NCOLS,), jnp.float32),
            pltpu.VMEM((HIST_U * L * NB,), jnp.int32),
            pltpu.VMEM((NB,), jnp.int32),
            pltpu.VMEM((TOPK,), jnp.float32),
        ],
        compiler_params=pltpu.CompilerParams(needs_layout_passes=False),
    )
    def body(x_hbm, out_hbm, row_v, cand_v, hist_v, tot_v, sort_v):
        cid = lax.axis_index("c")
        sid = lax.axis_index("s")
        wid = sid * 2 + cid

        lane = lax.iota(jnp.int32, L)
        lane_base = lane * NB
        ones = jnp.ones((L,), jnp.int32)
        zeros = jnp.zeros((L,), jnp.int32)

        def to_key(v):
            b = lax.bitcast_convert_type(v, jnp.int32)
            sgn = b >> 31
            return lax.bitcast_convert_type(b ^ (sgn | jnp.int32(-(2 ** 31))), jnp.uint32)

        @plsc.parallel_loop(0, HIST_U * NB, unroll=8)
        def _zh(i):
            hist_v[pl.ds(i * L, L)] = zeros

        def digit_select(target_v, banks):
            @plsc.parallel_loop(0, NB // L, unroll=2)
            def _lr(blk):
                acc = zeros
                for l in range(L * banks):
                    seg = pl.ds(l * NB + blk * L, L)
                    acc = acc + hist_v[seg]
                    hist_v[seg] = zeros
                tot_v[pl.ds(blk * L, L)] = acc

            s_blk = zeros
            for kk in range(L):
                s_blk = s_blk + plsc.load_gather(tot_v, [lane * L + kk])
            rs = lax.rev(s_blk, (0,))
            rc = plsc.cumsum(rs)
            fb = plsc.all_reduce_ffs(rc >= target_v)
            above_b = rc[fb] - rs[fb]
            blk_sel = (L - 1) - fb
            t2 = plsc.load_gather(tot_v, [blk_sel * L + lane])
            t2r = lax.rev(t2, (0,))
            c2 = plsc.cumsum(t2r) + above_b
            f2 = plsc.all_reduce_ffs(c2 >= target_v)
            above = c2[f2] - t2r[f2]
            bsel = blk_sel * L + (L - 1) - f2
            return bsel, above

        @pl.loop(0, ROWS_PER_W)
        def _row(rl):
            r = wid * ROWS_PER_W + rl
            pltpu.sync_copy(x_hbm.at[r], row_v)

            @plsc.parallel_loop(0, NV_ROW, unroll=8)
            def _h1(i):
                b = lax.bitcast_convert_type(row_v[pl.ds(i * L, L)], jnp.int32)
                dig = (b >> 24) ^ ((b >> 31) | jnp.int32(0x80))
                bank = (i & (HIST_U - 1)) * (L * NB)
                plsc.addupdate_scatter(
                    hist_v, [jnp.broadcast_to(bank, (L,)) + lane_base + dig], ones)

            topk_v = jnp.broadcast_to(jnp.int32(TOPK), (L,))
            b1, n_hi = digit_select(topk_v, HIST_U)

            thr_u = b1.astype(jnp.uint32) << jnp.uint32(24)
            thr_bits = jnp.where(thr_u >= jnp.uint32(0x80000000),
                                 thr_u ^ jnp.uint32(0x80000000), ~thr_u)
            thr_f = lax.bitcast_convert_type(thr_bits, jnp.float32)
            thr_f = jnp.where(b1 == 0, jnp.float32(-jnp.inf), thr_f)

            @plsc.parallel_loop(0, NV_ROW, unroll=8, carry=zeros)
            def compact(i, cnt):
                v = row_v[pl.ds(i * L, L)]
                mk = v >= thr_f
                plsc.store_scatter(cand_v, [(cnt << 4) + lane], v, mask=mk)
                return cnt + jnp.where(mk, 1, 0)

            cnt_v = compact
            mx = jnp.max(cnt_v)

            kth = topk_v - n_hi
            tkey = b1.astype(jnp.uint32) << jnp.uint32(24)
            for s in (16, 8, 0):
                tpref_v = tkey >> jnp.uint32(s + 8)

                @plsc.parallel_loop(0, mx, unroll=REF_U)
                def _hr(i):
                    u = to_key(cand_v[pl.ds(i * L, L)])
                    valid = jnp.broadcast_to(i, (L,)) < cnt_v
                    mk = jnp.logical_and(valid, (u >> jnp.uint32(s + 8)) == tpref_v)
                    dig = ((u >> jnp.uint32(s)) & jnp.uint32(0xFF)).astype(jnp.int32)
                    bank = (i & (REF_U - 1)) * (L * NB)
                    plsc.addupdate_scatter(
                        hist_v, [jnp.broadcast_to(bank, (L,)) + lane_base + dig],
                        ones, mask=mk)

                bl, above = digit_select(kth, REF_U)
                tkey = tkey | (bl.astype(jnp.uint32) << jnp.uint32(s))
                kth = kth - above

            tbits = jnp.where(tkey >= jnp.uint32(0x80000000),
                              tkey ^ jnp.uint32(0x80000000), ~tkey)
            t_f = lax.bitcast_convert_type(tbits, jnp.float32)

            @plsc.parallel_loop(0, NV_K, unroll=8)
            def _fill(i):
                sort_v[pl.ds(i * L, L)] = t_f

            @plsc.parallel_loop(0, mx, unroll=2, carry=zeros)
            def _fc(i, offv):
                v = cand_v[pl.ds(i * L, L)]
                valid = jnp.broadcast_to(i, (L,)) < cnt_v
                mk = jnp.logical_and(v > t_f, valid)
                pos = offv + plsc.cumsum(jnp.where(mk, 1, 0)) - 1
                plsc.store_scatter(sort_v, [pos], v, mask=mk)
                return offv + plsc.all_reduce_population_count(mk)

            def vsort_sweep():
                @plsc.parallel_loop(0, NV_K, unroll=8)
                def _vs(i):
                    v = sort_v[pl.ds(i * L, L)]
                    sk, _ = plsc.sort_key_val(v, v, descending=True)
                    sort_v[pl.ds(i * L, L)] = sk

            vsort_sweep()
            run = 1
            while run < NV_K:
                lg = run.bit_length() - 1

                @plsc.parallel_loop(0, NV_K // 2, unroll=8)
                def _ma(t, run=run, lg=lg):
                    base = (t >> lg) * (2 * run)
                    i = t & (run - 1)
                    p = (base + i) * L
                    q = (base + 2 * run - 1 - i) * L
                    a = sort_v[pl.ds(p, L)]
                    b = lax.rev(sort_v[pl.ds(q, L)], (0,))
                    sort_v[pl.ds(p, L)] = jnp.maximum(a, b)
                    sort_v[pl.ds(q, L)] = lax.rev(jnp.minimum(a, b), (0,))

                d = run // 2
                while d >= 1:
                    lgd = d.bit_length() - 1

                    @plsc.parallel_loop(0, NV_K // 2, unroll=8)
                    def _mb(t, d=d, lgd=lgd):
                        p = ((t >> lgd) * (2 * d) + (t & (d - 1))) * L
                        q = p + d * L
                        a = sort_v[pl.ds(p, L)]
                        b = sort_v[pl.ds(q, L)]
                        sort_v[pl.ds(p, L)] = jnp.maximum(a, b)
                        sort_v[pl.ds(q, L)] = jnp.minimum(a, b)

                    d //= 2
                vsort_sweep()
                run *= 2

            pltpu.sync_copy(sort_v, out_hbm.at[r])

    return body(x)
